# Initial kernel scaffold; baseline (speedup 1.0000x reference)
import functools

import jax
import jax.numpy as jnp
import numpy as np
from jax.experimental import pallas as pl
from jax.experimental.pallas import tpu as pltpu

VAR0 = 0.1
VAR1 = 0.2
SIGMA_REPGT = 0.9
EPS = 1e-10
LOG1MS = np.float32(np.log(1.0 - SIGMA_REPGT))

TN = 256


def _decode_cols(l4n, p4n):
    lx, ly, lw, lh = l4n[0:1, :], l4n[1:2, :], l4n[2:3, :], l4n[3:4, :]
    px, py, pw, ph = p4n[0:1, :], p4n[1:2, :], p4n[2:3, :], p4n[3:4, :]
    cx = px + lx * VAR0 * pw
    cy = py + ly * VAR0 * ph
    w = pw * jnp.exp(lw * VAR1)
    h = ph * jnp.exp(lh * VAR1)
    x1 = cx - w * 0.5
    y1 = cy - h * 0.5
    x2 = cx + w * 0.5
    y2 = cy + h * 0.5
    area = (x2 - x1) * (y2 - y1)
    return x1, y1, x2, y2, area


def _decode_rows(lr, pr):
    lx, ly, lw, lh = lr[:, 0:1], lr[:, 1:2], lr[:, 2:3], lr[:, 3:4]
    px, py, pw, ph = pr[:, 0:1], pr[:, 1:2], pr[:, 2:3], pr[:, 3:4]
    cx = px + lx * VAR0 * pw
    cy = py + ly * VAR0 * ph
    w = pw * jnp.exp(lw * VAR1)
    h = ph * jnp.exp(lh * VAR1)
    x1 = cx - w * 0.5
    y1 = cy - h * 0.5
    x2 = cx + w * 0.5
    y2 = cy + h * 0.5
    area = (x2 - x1) * (y2 - y1)
    return x1, y1, x2, y2, area


def _rep_kernel(n_tiles, g, loc_r, pri_r, m_row, loc_c, pri_c, m_col, gt_c,
                out_ref):
    t = pl.program_id(1)

    x1r, y1r, x2r, y2r, area_r = _decode_rows(loc_r[0], pri_r[...])
    mr = m_row[0]
    mrb = mr > 0.0

    x1c, y1c, x2c, y2c, area_c = _decode_cols(loc_c[0], pri_c[...])
    mcb = m_col[0] > 0.0

    iw = jnp.maximum(jnp.minimum(x2r, x2c) - jnp.maximum(x1r, x1c), 0.0)
    ih = jnp.maximum(jnp.minimum(y2r, y2c) - jnp.maximum(y1r, y1c), 0.0)
    inter = iw * ih
    ov = inter / (area_r + area_c - inter)
    valid = mrb & mcb & (ov > 0.0)
    tb = jnp.sum(jnp.where(valid, ov, 0.0))
    nb = jnp.sum(jnp.where(valid, 1.0, 0.0))

    gt = gt_c[0]
    gx1, gy1, gx2, gy2 = gt[0:1, :], gt[1:2, :], gt[2:3, :], gt[3:4, :]
    garea = (gx2 - gx1) * (gy2 - gy1)
    giw = jnp.maximum(jnp.minimum(x2r, gx2) - jnp.maximum(x1r, gx1), 0.0)
    gih = jnp.maximum(jnp.minimum(y2r, gy2) - jnp.maximum(y1r, gy1), 0.0)
    ginter = giw * gih
    gov = (ginter / (area_r + garea - ginter)) * mr

    cols = jax.lax.broadcasted_iota(jnp.int32, (TN, g), 1)
    max1 = jnp.max(gov, axis=1, keepdims=True)
    arg1 = jnp.min(jnp.where(gov == max1, cols, g), axis=1, keepdims=True)
    ov2 = jnp.where(cols == arg1, 0.0, gov)
    max2 = jnp.max(ov2, axis=1, keepdims=True)
    arg2 = jnp.min(jnp.where(ov2 == max2, cols, g), axis=1, keepdims=True)
    onehot2 = jnp.where(cols == arg2, 1.0, 0.0)

    def sel(v):
        return jnp.sum(onehot2 * v, axis=1, keepdims=True)

    sx1, sy1, sx2, sy2 = sel(gx1), sel(gy1), sel(gx2), sel(gy2)
    sarea = sel(garea)
    iw2 = jnp.maximum(jnp.minimum(x2r, sx2) - jnp.maximum(x1r, sx1), 0.0)
    ih2 = jnp.maximum(jnp.minimum(y2r, sy2) - jnp.maximum(y1r, sy1), 0.0)
    iog = (iw2 * ih2) / sarea
    iog_safe = jnp.where(iog > SIGMA_REPGT, 0.0, iog)
    term = jnp.where(iog > SIGMA_REPGT,
                     (iog - SIGMA_REPGT) / (1.0 - SIGMA_REPGT) - LOG1MS,
                     -jnp.log(jnp.maximum(1.0 - iog_safe, EPS)))
    contrib = (max2 > 0.0) & mrb
    tg = jnp.sum(jnp.where(contrib, term, 0.0))
    ng = jnp.sum(jnp.where(contrib, 1.0, 0.0))

    lane = jax.lax.broadcasted_iota(jnp.int32, (1, 128), 1)
    upd = (jnp.where(lane == 0, tg, 0.0) + jnp.where(lane == 1, ng, 0.0) +
           jnp.where(lane == 2, tb, 0.0) + jnp.where(lane == 3, nb, 0.0))

    @pl.when(t == 0)
    def _():
        out_ref[...] = jnp.zeros_like(out_ref)

    out_ref[...] += upd

    @pl.when(t == n_tiles - 1)
    def _():
        a = out_ref[...]
        tg_, ng_, tb_, nb_ = a[0, 0], a[0, 1], a[0, 2], a[0, 3]
        lgt = jnp.where(ng_ > 0.0, tg_ / jnp.maximum(ng_, 1.0), 0.0)
        lbx = jnp.where(nb_ > 0.0, tb_ / jnp.maximum(nb_, 1.0), 0.0)
        out_ref[...] = jnp.where(lane == 0, lgt + lbx, 0.0)


@jax.jit
def kernel(loc_data, ground_data, prior_data, pos_idx):
    b, n, _ = loc_data.shape
    g = ground_data.shape[1]
    n_tiles = n // TN

    maskf = pos_idx[..., 0].astype(jnp.float32)
    loc_col = jnp.transpose(loc_data, (0, 2, 1))
    prior_col = prior_data.T
    gt_col = jnp.transpose(ground_data, (0, 2, 1))
    mask_col = maskf.reshape(b, 1, n)
    mask_row = maskf.reshape(b, n, 1)

    out = pl.pallas_call(
        functools.partial(_rep_kernel, n_tiles, g),
        out_shape=jax.ShapeDtypeStruct((b, 128), jnp.float32),
        grid=(b, n_tiles),
        in_specs=[
            pl.BlockSpec((1, TN, 4), lambda i, j: (i, j, 0)),
            pl.BlockSpec((TN, 4), lambda i, j: (j, 0)),
            pl.BlockSpec((1, TN, 1), lambda i, j: (i, j, 0)),
            pl.BlockSpec((1, 4, n), lambda i, j: (i, 0, 0)),
            pl.BlockSpec((4, n), lambda i, j: (0, 0)),
            pl.BlockSpec((1, 1, n), lambda i, j: (i, 0, 0)),
            pl.BlockSpec((1, 4, g), lambda i, j: (i, 0, 0)),
        ],
        out_specs=pl.BlockSpec((1, 128), lambda i, j: (i, 0)),
        compiler_params=pltpu.CompilerParams(
            dimension_semantics=("core_parallel", "arbitrary"),
        ),
        name="repulsion_loss",
    )(loc_data, prior_data, mask_row, loc_col, prior_col, mask_col, gt_col)

    return jnp.sum(out[:, 0])


# speedup vs baseline: 2.5737x; 2.5737x over previous
import functools

import jax
import jax.numpy as jnp
import numpy as np
from jax.experimental import pallas as pl
from jax.experimental.pallas import tpu as pltpu

VAR0 = 0.1
VAR1 = 0.2
SIGMA_REPGT = 0.9
EPS = 1e-10
LOG1MS = np.float32(np.log(1.0 - SIGMA_REPGT))

TN = 256


def _decode_cols(l4n, p4n):
    lx, ly, lw, lh = l4n[0:1, :], l4n[1:2, :], l4n[2:3, :], l4n[3:4, :]
    px, py, pw, ph = p4n[0:1, :], p4n[1:2, :], p4n[2:3, :], p4n[3:4, :]
    cx = px + lx * VAR0 * pw
    cy = py + ly * VAR0 * ph
    w = pw * jnp.exp(lw * VAR1)
    h = ph * jnp.exp(lh * VAR1)
    x1 = cx - w * 0.5
    y1 = cy - h * 0.5
    x2 = cx + w * 0.5
    y2 = cy + h * 0.5
    area = (x2 - x1) * (y2 - y1)
    return x1, y1, x2, y2, area


def _decode_rows(lr, pr):
    lx, ly, lw, lh = lr[:, 0:1], lr[:, 1:2], lr[:, 2:3], lr[:, 3:4]
    px, py, pw, ph = pr[:, 0:1], pr[:, 1:2], pr[:, 2:3], pr[:, 3:4]
    cx = px + lx * VAR0 * pw
    cy = py + ly * VAR0 * ph
    w = pw * jnp.exp(lw * VAR1)
    h = ph * jnp.exp(lh * VAR1)
    x1 = cx - w * 0.5
    y1 = cy - h * 0.5
    x2 = cx + w * 0.5
    y2 = cy + h * 0.5
    area = (x2 - x1) * (y2 - y1)
    return x1, y1, x2, y2, area


def _rep_kernel(n_tiles, g, loc_r, pri_r, m_row, loc_c, pri_c, m_col, gt_c,
                out_ref):
    t = pl.program_id(1)

    x1r, y1r, x2r, y2r, area_r = _decode_rows(loc_r[0], pri_r[...])
    mr = m_row[0]
    mrb = mr > 0.0

    x1c, y1c, x2c, y2c, area_c = _decode_cols(loc_c[0], pri_c[...])
    mcb = m_col[0] > 0.0

    iw = jnp.maximum(jnp.minimum(x2r, x2c) - jnp.maximum(x1r, x1c), 0.0)
    ih = jnp.maximum(jnp.minimum(y2r, y2c) - jnp.maximum(y1r, y1c), 0.0)
    inter = iw * ih
    ov = inter / (area_r + area_c - inter)
    valid = mrb & mcb & (ov > 0.0)
    tb = jnp.sum(jnp.where(valid, ov, 0.0))
    nb = jnp.sum(jnp.where(valid, 1.0, 0.0))

    gt = gt_c[0]
    gx1, gy1, gx2, gy2 = gt[0:1, :], gt[1:2, :], gt[2:3, :], gt[3:4, :]
    garea = (gx2 - gx1) * (gy2 - gy1)
    giw = jnp.maximum(jnp.minimum(x2r, gx2) - jnp.maximum(x1r, gx1), 0.0)
    gih = jnp.maximum(jnp.minimum(y2r, gy2) - jnp.maximum(y1r, gy1), 0.0)
    ginter = giw * gih
    gov = (ginter / (area_r + garea - ginter)) * mr

    cols = jax.lax.broadcasted_iota(jnp.int32, (TN, g), 1)
    max1 = jnp.max(gov, axis=1, keepdims=True)
    arg1 = jnp.min(jnp.where(gov == max1, cols, g), axis=1, keepdims=True)
    ov2 = jnp.where(cols == arg1, 0.0, gov)
    max2 = jnp.max(ov2, axis=1, keepdims=True)
    arg2 = jnp.min(jnp.where(ov2 == max2, cols, g), axis=1, keepdims=True)
    onehot2 = jnp.where(cols == arg2, 1.0, 0.0)

    def sel(v):
        return jnp.sum(onehot2 * v, axis=1, keepdims=True)

    sx1, sy1, sx2, sy2 = sel(gx1), sel(gy1), sel(gx2), sel(gy2)
    sarea = sel(garea)
    iw2 = jnp.maximum(jnp.minimum(x2r, sx2) - jnp.maximum(x1r, sx1), 0.0)
    ih2 = jnp.maximum(jnp.minimum(y2r, sy2) - jnp.maximum(y1r, sy1), 0.0)
    iog = (iw2 * ih2) / sarea
    iog_safe = jnp.where(iog > SIGMA_REPGT, 0.0, iog)
    term = jnp.where(iog > SIGMA_REPGT,
                     (iog - SIGMA_REPGT) / (1.0 - SIGMA_REPGT) - LOG1MS,
                     -jnp.log(jnp.maximum(1.0 - iog_safe, EPS)))
    contrib = (max2 > 0.0) & mrb
    tg = jnp.sum(jnp.where(contrib, term, 0.0))
    ng = jnp.sum(jnp.where(contrib, 1.0, 0.0))

    lane = jax.lax.broadcasted_iota(jnp.int32, (1, 1, 128), 2)
    upd = (jnp.where(lane == 0, tg, 0.0) + jnp.where(lane == 1, ng, 0.0) +
           jnp.where(lane == 2, tb, 0.0) + jnp.where(lane == 3, nb, 0.0))

    @pl.when(t == 0)
    def _():
        out_ref[...] = jnp.zeros_like(out_ref)

    out_ref[...] += upd

    @pl.when(t == n_tiles - 1)
    def _():
        a = out_ref[...]
        tg_, ng_, tb_, nb_ = a[0, 0, 0], a[0, 0, 1], a[0, 0, 2], a[0, 0, 3]
        lgt = jnp.where(ng_ > 0.0, tg_ / jnp.maximum(ng_, 1.0), 0.0)
        lbx = jnp.where(nb_ > 0.0, tb_ / jnp.maximum(nb_, 1.0), 0.0)
        out_ref[...] = jnp.where(lane == 0, lgt + lbx, 0.0)


@jax.jit
def kernel(loc_data, ground_data, prior_data, pos_idx):
    b, n, _ = loc_data.shape
    g = ground_data.shape[1]
    n_tiles = n // TN

    maskf = pos_idx[..., 0].astype(jnp.float32)
    loc_col = jnp.transpose(loc_data, (0, 2, 1))
    prior_col = prior_data.T
    gt_col = jnp.transpose(ground_data, (0, 2, 1))
    mask_col = maskf.reshape(b, 1, n)
    mask_row = maskf.reshape(b, n, 1)

    out = pl.pallas_call(
        functools.partial(_rep_kernel, n_tiles, g),
        out_shape=jax.ShapeDtypeStruct((b, 1, 128), jnp.float32),
        grid=(b, n_tiles),
        in_specs=[
            pl.BlockSpec((1, TN, 4), lambda i, j: (i, j, 0)),
            pl.BlockSpec((TN, 4), lambda i, j: (j, 0)),
            pl.BlockSpec((1, TN, 1), lambda i, j: (i, j, 0)),
            pl.BlockSpec((1, 4, n), lambda i, j: (i, 0, 0)),
            pl.BlockSpec((4, n), lambda i, j: (0, 0)),
            pl.BlockSpec((1, 1, n), lambda i, j: (i, 0, 0)),
            pl.BlockSpec((1, 4, g), lambda i, j: (i, 0, 0)),
        ],
        out_specs=pl.BlockSpec((1, 1, 128), lambda i, j: (i, 0, 0)),
        compiler_params=pltpu.CompilerParams(
            dimension_semantics=("parallel", "arbitrary"),
        ),
        name="repulsion_loss",
    )(loc_data, prior_data, mask_row, loc_col, prior_col, mask_col, gt_col)

    return jnp.sum(out[:, 0, 0])
